# baseline (speedup 1.0000x reference)
import functools
import jax
import jax.numpy as jnp
from jax import lax
from jax.experimental import pallas as pl
from jax.experimental.pallas import tpu as pltpu
from jax.experimental.pallas import tpu_sc as plsc

_N = 10000
_E = 320000
_M = 80000
_IN = 128
_HID = 256

_F = 16
_NB = _HID // _F
_NCORE = 2
_NSUB = 16
_BATCH = 512
_JTILE = 40
_EPAD = _JTILE * _NSUB * _BATCH
_ME = 81920
_CH = 512
_ECH = _ME // _NSUB // _CH
_NV = 10112
_NP = 10016


def _mm_kernel(x_ref, w_ref, o_ref):
    o_ref[...] = jnp.dot(x_ref[...], w_ref[...],
                         preferred_element_type=jnp.float32)


def _matmul(X, W):
    BM = 1000
    return pl.pallas_call(
        _mm_kernel,
        grid=(_N // BM,),
        in_specs=[
            pl.BlockSpec((BM, _IN), lambda i: (i, 0)),
            pl.BlockSpec((_IN, _HID), lambda i: (0, 0)),
        ],
        out_specs=pl.BlockSpec((BM, _HID), lambda i: (i, 0)),
        out_shape=jax.ShapeDtypeStruct((_N, _HID), jnp.float32),
    )(X, W)


def _ep_kernel(eps_ref, xh_ref, xv_ref, o_ref):
    o = (1.0 + eps_ref[0]) * xh_ref[...] + xv_ref[...]
    ss = jnp.sum(o * o, axis=1, keepdims=True)
    rn = jnp.sqrt(ss)
    scale = jnp.where(rn > 0, 1.0 / rn, 0.0)
    o_ref[...] = o * scale


def _epilogue(eps, Xh, Xv):
    BM = 1000
    return pl.pallas_call(
        _ep_kernel,
        grid=(_N // BM,),
        in_specs=[
            pl.BlockSpec(memory_space=pltpu.SMEM),
            pl.BlockSpec((BM, _HID), lambda i: (i, 0)),
            pl.BlockSpec((BM, _HID), lambda i: (i, 0)),
        ],
        out_specs=pl.BlockSpec((BM, _HID), lambda i: (i, 0)),
        out_shape=jax.ShapeDtypeStruct((_N, _HID), jnp.float32),
    )(eps, Xh, Xv)


def _sc_body(xh, v2d, e2d, zsrc, osrc, out, invc, xe,
             acc_e, acc_v, ivb, ieb, rows, sbuf, jbuf, sem_g, sem_s):
    c = lax.axis_index("c")
    s = lax.axis_index("s")

    ebase = s * (_ME // _NSUB)
    jbase = s * _JTILE

    def zero_acc_e():
        pltpu.sync_copy(zsrc, sbuf)
        for k in range(_ECH):
            pltpu.sync_copy(sbuf, acc_e.at[pl.ds(ebase + k * _CH, _CH)])

    zero_acc_e()
    plsc.subcore_barrier()

    pltpu.sync_copy(osrc, rows.at[0])

    def cbody(j, carry):
        pltpu.sync_copy(e2d.at[jbase + j], ieb.at[0])
        pltpu.sync_copy(rows.at[0], acc_e.at[ieb.at[0]], add=True)
        return carry
    lax.fori_loop(0, _JTILE, cbody, 0)
    plsc.subcore_barrier()

    def run_pass(gtable, gidx_hbm, sidx_hbm, gidx, sidx, dacc):
        def load_idx(j, b):
            pltpu.sync_copy(gidx_hbm.at[jbase + j], gidx.at[b])
            pltpu.sync_copy(sidx_hbm.at[jbase + j], sidx.at[b])

        def issue_g(b):
            pltpu.async_copy(gtable.at[gidx.at[b]], rows.at[b], sem_g)

        def wait_g(b):
            pltpu.make_async_copy(gtable.at[gidx.at[b]], rows.at[b],
                                  sem_g).wait()

        def issue_s(b):
            pltpu.async_copy(rows.at[b], dacc.at[sidx.at[b]], sem_s, add=True)

        def wait_s(b):
            pltpu.make_async_copy(rows.at[b], dacc.at[sidx.at[b]],
                                  sem_s).wait()

        load_idx(0, 0)
        issue_g(0)
        load_idx(1, 1)
        issue_g(1)
        wait_g(0)
        issue_s(0)

        def pbody(jj, carry):
            for b in (0, 1):
                j = jj * 2 + b
                wait_s(b)
                load_idx(j, b)
                issue_g(b)
                wait_g(1 - b)
                issue_s(1 - b)
            return carry
        lax.fori_loop(1, _JTILE // 2, pbody, 0)

        wait_g(1)
        issue_s(1)
        wait_s(0)
        wait_s(1)

    for k in range(_ECH):
        pltpu.sync_copy(acc_e.at[pl.ds(ebase + k * _CH, _CH)], sbuf)

        def gbody(r, carry):
            sbuf[r, :] = 1.0 / jnp.maximum(sbuf[r, :], 1.0)
            return carry
        lax.fori_loop(0, _CH, gbody, 0)
        pltpu.sync_copy(sbuf, invc.at[c].at[pl.ds(ebase + k * _CH, _CH)])
    plsc.subcore_barrier()

    for bl in range(_NB // _NCORE):
        bg = c * (_NB // _NCORE) + bl
        zero_acc_e()
        pltpu.sync_copy(sbuf, acc_v.at[pl.ds(s * 632, _CH)])
        pltpu.sync_copy(sbuf.at[pl.ds(0, 120)],
                        acc_v.at[pl.ds(s * 632 + _CH, 120)])
        plsc.subcore_barrier()

        run_pass(xh.at[bg], v2d, e2d, ivb, ieb, acc_e)
        plsc.subcore_barrier()

        for k in range(_ECH):
            pltpu.sync_copy(acc_e.at[pl.ds(ebase + k * _CH, _CH)], sbuf)
            pltpu.sync_copy(invc.at[c].at[pl.ds(ebase + k * _CH, _CH)], jbuf)

            def scbody(r, carry):
                sbuf[r, :] = sbuf[r, :] * jbuf[r, :]
                return carry
            lax.fori_loop(0, _CH, scbody, 0)
            pltpu.sync_copy(sbuf, xe.at[c].at[pl.ds(ebase + k * _CH, _CH)])
        plsc.subcore_barrier()

        run_pass(xe.at[c], e2d, v2d, ieb, ivb, acc_v)
        plsc.subcore_barrier()

        @pl.when(s < _NSUB - 1)
        def _():
            pltpu.sync_copy(acc_v.at[pl.ds(s * 624, 624)],
                            out.at[bg].at[pl.ds(s * 624, 624)])

        @pl.when(s == _NSUB - 1)
        def _():
            pltpu.sync_copy(acc_v.at[pl.ds(15 * 624, 640)],
                            out.at[bg].at[pl.ds(15 * 624, 640)])
        plsc.subcore_barrier()


_sc_call = pl.kernel(
    _sc_body,
    out_type=(
        jax.ShapeDtypeStruct((_NB, _N, _F), jnp.float32),
        jax.ShapeDtypeStruct((_NCORE, _ME, _F), jnp.float32),
        jax.ShapeDtypeStruct((_NCORE, _ME, _F), jnp.float32),
    ),
    mesh=plsc.VectorSubcoreMesh(core_axis_name="c", subcore_axis_name="s"),
    compiler_params=pltpu.CompilerParams(use_tc_tiling_on_sc=False),
    scratch_types=[
        pltpu.VMEM_SHARED((_ME, _F), jnp.float32),
        pltpu.VMEM_SHARED((_NV, _F), jnp.float32),
        pltpu.VMEM((2, _BATCH), jnp.int32),
        pltpu.VMEM((2, _BATCH), jnp.int32),
        pltpu.VMEM((2, _BATCH, _F), jnp.float32),
        pltpu.VMEM((_CH, _F), jnp.float32),
        pltpu.VMEM((_CH, _F), jnp.float32),
        pltpu.SemaphoreType.DMA,
        pltpu.SemaphoreType.DMA,
    ],
)


def kernel(X, vertex, edges, W, eps):
    Xh = _matmul(X, W)

    xh_pad = jnp.concatenate(
        [Xh, jnp.zeros((_NP - _N, _HID), jnp.float32)], axis=0)
    xh_b = xh_pad.reshape(_NP, _NB, _F).transpose(1, 0, 2)

    pad = _EPAD - _E
    v2d = jnp.concatenate(
        [vertex.astype(jnp.int32), jnp.full((pad,), _N, jnp.int32)]
    ).reshape(_EPAD // _BATCH, _BATCH)
    e2d = jnp.concatenate(
        [edges.astype(jnp.int32), jnp.full((pad,), _M, jnp.int32)]
    ).reshape(_EPAD // _BATCH, _BATCH)

    zsrc = jnp.zeros((_CH, _F), jnp.float32)
    osrc = jnp.ones((_BATCH, _F), jnp.float32)

    Xv_b, _unused_invc, _unused_xe = _sc_call(xh_b, v2d, e2d, zsrc, osrc)
    Xv = Xv_b.transpose(1, 0, 2).reshape(_N, _HID)
    return _epilogue(eps, Xh, Xv)


# speedup vs baseline: 1.1905x; 1.1905x over previous
import functools
import jax
import jax.numpy as jnp
from jax import lax
from jax.experimental import pallas as pl
from jax.experimental.pallas import tpu as pltpu
from jax.experimental.pallas import tpu_sc as plsc

_N = 10000
_E = 320000
_M = 80000
_IN = 128
_HID = 256

_F = 16
_NB = _HID // _F
_NCORE = 2
_NSUB = 16
_BATCH = 512
_JTILE = 40
_EPAD = _JTILE * _NSUB * _BATCH
_ME = 81920
_CH = 256
_ECH = _ME // _NSUB // _CH
_NV = 10112
_NP = 10016


def _mm_kernel(x_ref, w_ref, o_ref):
    o_ref[...] = jnp.dot(x_ref[...], w_ref[...],
                         preferred_element_type=jnp.float32)


def _matmul(X, W):
    BM = 1000
    return pl.pallas_call(
        _mm_kernel,
        grid=(_N // BM,),
        in_specs=[
            pl.BlockSpec((BM, _IN), lambda i: (i, 0)),
            pl.BlockSpec((_IN, _HID), lambda i: (0, 0)),
        ],
        out_specs=pl.BlockSpec((BM, _HID), lambda i: (i, 0)),
        out_shape=jax.ShapeDtypeStruct((_N, _HID), jnp.float32),
    )(X, W)


def _ep_kernel(eps_ref, xh_ref, xv_ref, o_ref):
    o = (1.0 + eps_ref[0]) * xh_ref[...] + xv_ref[...]
    ss = jnp.sum(o * o, axis=1, keepdims=True)
    rn = jnp.sqrt(ss)
    scale = jnp.where(rn > 0, 1.0 / rn, 0.0)
    o_ref[...] = o * scale


def _epilogue(eps, Xh, Xv):
    BM = 1000
    return pl.pallas_call(
        _ep_kernel,
        grid=(_N // BM,),
        in_specs=[
            pl.BlockSpec(memory_space=pltpu.SMEM),
            pl.BlockSpec((BM, _HID), lambda i: (i, 0)),
            pl.BlockSpec((BM, _HID), lambda i: (i, 0)),
        ],
        out_specs=pl.BlockSpec((BM, _HID), lambda i: (i, 0)),
        out_shape=jax.ShapeDtypeStruct((_N, _HID), jnp.float32),
    )(eps, Xh, Xv)


def _sc_body(xh, v2d, e2d, zsrc, osrc, out, invc,
             acc_e, acc_v, tbl, ivb, ieb, rows, sbuf, jbuf, sem_g, sem_s):
    c = lax.axis_index("c")
    s = lax.axis_index("s")

    ebase = s * (_ME // _NSUB)
    jbase = s * _JTILE

    def zero_acc_e():
        pltpu.sync_copy(zsrc, sbuf)
        for k in range(_ECH):
            pltpu.sync_copy(sbuf, acc_e.at[pl.ds(ebase + k * _CH, _CH)])

    zero_acc_e()
    plsc.subcore_barrier()

    pltpu.sync_copy(osrc, rows.at[0])

    def cbody(j, carry):
        pltpu.sync_copy(e2d.at[jbase + j], ieb.at[0])
        pltpu.sync_copy(rows.at[0], acc_e.at[ieb.at[0]], add=True)
        return carry
    lax.fori_loop(0, _JTILE, cbody, 0)
    plsc.subcore_barrier()

    def run_pass(gtable, gidx_hbm, sidx_hbm, gidx, sidx, dacc):
        def load_idx(j, b):
            pltpu.sync_copy(gidx_hbm.at[jbase + j], gidx.at[b])
            pltpu.sync_copy(sidx_hbm.at[jbase + j], sidx.at[b])

        def issue_g(b):
            pltpu.async_copy(gtable.at[gidx.at[b]], rows.at[b], sem_g)

        def wait_g(b):
            pltpu.make_async_copy(gtable.at[gidx.at[b]], rows.at[b],
                                  sem_g).wait()

        def issue_s(b):
            pltpu.async_copy(rows.at[b], dacc.at[sidx.at[b]], sem_s, add=True)

        def wait_s(b):
            pltpu.make_async_copy(rows.at[b], dacc.at[sidx.at[b]],
                                  sem_s).wait()

        load_idx(0, 0)
        issue_g(0)
        load_idx(1, 1)
        issue_g(1)
        wait_g(0)
        issue_s(0)

        def pbody(jj, carry):
            for b in (0, 1):
                j = jj * 2 + b
                wait_s(b)
                load_idx(j, b)
                issue_g(b)
                wait_g(1 - b)
                issue_s(1 - b)
            return carry
        lax.fori_loop(1, _JTILE // 2, pbody, 0)

        wait_g(1)
        issue_s(1)
        wait_s(0)
        wait_s(1)

    for k in range(_ECH):
        pltpu.sync_copy(acc_e.at[pl.ds(ebase + k * _CH, _CH)], sbuf)

        def gbody(r, carry):
            sbuf[r, :] = 1.0 / jnp.maximum(sbuf[r, :], 1.0)
            return carry
        lax.fori_loop(0, _CH, gbody, 0)
        pltpu.sync_copy(sbuf, invc.at[c].at[pl.ds(ebase + k * _CH, _CH)])
    plsc.subcore_barrier()

    for bl in range(_NB // _NCORE):
        bg = c * (_NB // _NCORE) + bl
        zero_acc_e()
        pltpu.sync_copy(sbuf, acc_v.at[pl.ds(s * 632, _CH)])
        pltpu.sync_copy(sbuf, acc_v.at[pl.ds(s * 632 + _CH, _CH)])
        pltpu.sync_copy(sbuf.at[pl.ds(0, 120)],
                        acc_v.at[pl.ds(s * 632 + 2 * _CH, 120)])

        @pl.when(s < _NSUB - 1)
        def _():
            pltpu.sync_copy(xh.at[bg].at[pl.ds(s * 632, 632)],
                            tbl.at[pl.ds(s * 632, 632)])

        @pl.when(s == _NSUB - 1)
        def _():
            pltpu.sync_copy(xh.at[bg].at[pl.ds(15 * 632, _NP - 15 * 632)],
                            tbl.at[pl.ds(15 * 632, _NP - 15 * 632)])
        plsc.subcore_barrier()

        run_pass(tbl, v2d, e2d, ivb, ieb, acc_e)
        plsc.subcore_barrier()

        for k in range(_ECH):
            pltpu.sync_copy(acc_e.at[pl.ds(ebase + k * _CH, _CH)], sbuf)
            pltpu.sync_copy(invc.at[c].at[pl.ds(ebase + k * _CH, _CH)], jbuf)

            def scbody(r, carry):
                sbuf[r, :] = sbuf[r, :] * jbuf[r, :]
                return carry
            lax.fori_loop(0, _CH, scbody, 0)
            pltpu.sync_copy(sbuf, acc_e.at[pl.ds(ebase + k * _CH, _CH)])
        plsc.subcore_barrier()

        run_pass(acc_e, e2d, v2d, ieb, ivb, acc_v)
        plsc.subcore_barrier()

        @pl.when(s < _NSUB - 1)
        def _():
            pltpu.sync_copy(acc_v.at[pl.ds(s * 624, 624)],
                            out.at[bg].at[pl.ds(s * 624, 624)])

        @pl.when(s == _NSUB - 1)
        def _():
            pltpu.sync_copy(acc_v.at[pl.ds(15 * 624, 640)],
                            out.at[bg].at[pl.ds(15 * 624, 640)])
        plsc.subcore_barrier()


_sc_call = pl.kernel(
    _sc_body,
    out_type=(
        jax.ShapeDtypeStruct((_NB, _N, _F), jnp.float32),
        jax.ShapeDtypeStruct((_NCORE, _ME, _F), jnp.float32),
    ),
    mesh=plsc.VectorSubcoreMesh(core_axis_name="c", subcore_axis_name="s"),
    compiler_params=pltpu.CompilerParams(use_tc_tiling_on_sc=False),
    scratch_types=[
        pltpu.VMEM_SHARED((_ME, _F), jnp.float32),
        pltpu.VMEM_SHARED((_NV, _F), jnp.float32),
        pltpu.VMEM_SHARED((_NP, _F), jnp.float32),
        pltpu.VMEM((2, _BATCH), jnp.int32),
        pltpu.VMEM((2, _BATCH), jnp.int32),
        pltpu.VMEM((2, _BATCH, _F), jnp.float32),
        pltpu.VMEM((_CH, _F), jnp.float32),
        pltpu.VMEM((_CH, _F), jnp.float32),
        pltpu.SemaphoreType.DMA,
        pltpu.SemaphoreType.DMA,
    ],
)


def kernel(X, vertex, edges, W, eps):
    Xh = _matmul(X, W)

    xh_pad = jnp.concatenate(
        [Xh, jnp.zeros((_NP - _N, _HID), jnp.float32)], axis=0)
    xh_b = xh_pad.reshape(_NP, _NB, _F).transpose(1, 0, 2)

    pad = _EPAD - _E
    v2d = jnp.concatenate(
        [vertex.astype(jnp.int32), jnp.full((pad,), _N, jnp.int32)]
    ).reshape(_EPAD // _BATCH, _BATCH)
    e2d = jnp.concatenate(
        [edges.astype(jnp.int32), jnp.full((pad,), _M, jnp.int32)]
    ).reshape(_EPAD // _BATCH, _BATCH)

    zsrc = jnp.zeros((_CH, _F), jnp.float32)
    osrc = jnp.ones((_BATCH, _F), jnp.float32)

    Xv_b, _unused_invc = _sc_call(xh_b, v2d, e2d, zsrc, osrc)
    Xv = Xv_b.transpose(1, 0, 2).reshape(_N, _HID)
    return _epilogue(eps, Xh, Xv)


# speedup vs baseline: 1.2375x; 1.0394x over previous
import functools
import jax
import jax.numpy as jnp
from jax import lax
from jax.experimental import pallas as pl
from jax.experimental.pallas import tpu as pltpu
from jax.experimental.pallas import tpu_sc as plsc

_N = 10000
_E = 320000
_M = 80000
_IN = 128
_HID = 256

_F = 16
_NB = _HID // _F
_NCORE = 2
_NSUB = 16
_BATCH = 512
_JTILE = 40
_EPAD = _JTILE * _NSUB * _BATCH
_ME = 81920
_CH = 256
_ECH = _ME // _NSUB // _CH
_NV = 10112
_NP = 10016


def _mm_kernel(x_ref, w_ref, o_ref):
    o_ref[...] = jnp.dot(x_ref[...], w_ref[...],
                         preferred_element_type=jnp.float32)


def _matmul(X, W):
    BM = 1000
    return pl.pallas_call(
        _mm_kernel,
        grid=(_N // BM,),
        in_specs=[
            pl.BlockSpec((BM, _IN), lambda i: (i, 0)),
            pl.BlockSpec((_IN, _HID), lambda i: (0, 0)),
        ],
        out_specs=pl.BlockSpec((BM, _HID), lambda i: (i, 0)),
        out_shape=jax.ShapeDtypeStruct((_N, _HID), jnp.float32),
    )(X, W)


def _ep_kernel(eps_ref, xh_ref, xv_ref, o_ref):
    o = (1.0 + eps_ref[0]) * xh_ref[...] + xv_ref[...]
    ss = jnp.sum(o * o, axis=1, keepdims=True)
    rn = jnp.sqrt(ss)
    scale = jnp.where(rn > 0, 1.0 / rn, 0.0)
    o_ref[...] = o * scale


def _epilogue(eps, Xh, Xv):
    BM = 1000
    return pl.pallas_call(
        _ep_kernel,
        grid=(_N // BM,),
        in_specs=[
            pl.BlockSpec(memory_space=pltpu.SMEM),
            pl.BlockSpec((BM, _HID), lambda i: (i, 0)),
            pl.BlockSpec((BM, _HID), lambda i: (i, 0)),
        ],
        out_specs=pl.BlockSpec((BM, _HID), lambda i: (i, 0)),
        out_shape=jax.ShapeDtypeStruct((_N, _HID), jnp.float32),
    )(eps, Xh, Xv)


def _sc_body(xh, v2d, e2d, zsrc, osrc, out, invc,
             acc_e, acc_v, tbl, ivb, ieb, rows, sbuf, jbuf, sem_g, sem_s):
    c = lax.axis_index("c")
    s = lax.axis_index("s")

    ebase = s * (_ME // _NSUB)
    jbase = s * _JTILE

    def zero_acc_e():
        pltpu.sync_copy(zsrc, sbuf)
        for k in range(_ECH):
            pltpu.async_copy(sbuf, acc_e.at[pl.ds(ebase + k * _CH, _CH)],
                             sem_g)

    def drain_zero_acc_e():
        for k in range(_ECH):
            pltpu.make_async_copy(sbuf, acc_e.at[pl.ds(ebase, _CH)],
                                  sem_g).wait()

    zero_acc_e()
    drain_zero_acc_e()
    plsc.subcore_barrier()

    pltpu.sync_copy(osrc, rows.at[0])
    pltpu.sync_copy(e2d.at[jbase], ieb.at[0])

    def cbody(jj, carry):
        for b in (0, 1):
            j = 2 * jj + b

            @pl.when(jj >= 1)
            def _():
                pltpu.make_async_copy(rows.at[0], acc_e.at[ieb.at[b]],
                                      sem_s).wait()
            pltpu.async_copy(rows.at[0], acc_e.at[ieb.at[b]], sem_s, add=True)

            @pl.when(j < _JTILE - 1)
            def _():
                pltpu.sync_copy(e2d.at[jbase + j + 1], ieb.at[1 - b])
        return carry
    lax.fori_loop(0, _JTILE // 2, cbody, 0)
    pltpu.make_async_copy(rows.at[0], acc_e.at[ieb.at[0]], sem_s).wait()
    pltpu.make_async_copy(rows.at[0], acc_e.at[ieb.at[1]], sem_s).wait()
    plsc.subcore_barrier()

    def run_pass(gtable, gidx_hbm, sidx_hbm, gidx, sidx, dacc):
        def load_idx(j, b):
            pltpu.sync_copy(gidx_hbm.at[jbase + j], gidx.at[b])
            pltpu.sync_copy(sidx_hbm.at[jbase + j], sidx.at[b])

        def issue_g(b):
            pltpu.async_copy(gtable.at[gidx.at[b]], rows.at[b], sem_g)

        def wait_g(b):
            pltpu.make_async_copy(gtable.at[gidx.at[b]], rows.at[b],
                                  sem_g).wait()

        def issue_s(b):
            pltpu.async_copy(rows.at[b], dacc.at[sidx.at[b]], sem_s, add=True)

        def wait_s(b):
            pltpu.make_async_copy(rows.at[b], dacc.at[sidx.at[b]],
                                  sem_s).wait()

        load_idx(0, 0)
        issue_g(0)
        load_idx(1, 1)
        issue_g(1)
        wait_g(0)
        issue_s(0)

        def pbody(jj, carry):
            for b in (0, 1):
                j = jj * 2 + b
                wait_s(b)
                load_idx(j, b)
                issue_g(b)
                wait_g(1 - b)
                issue_s(1 - b)
            return carry
        lax.fori_loop(1, _JTILE // 2, pbody, 0)

        wait_g(1)
        issue_s(1)
        wait_s(0)
        wait_s(1)

    for k in range(_ECH):
        pltpu.sync_copy(acc_e.at[pl.ds(ebase + k * _CH, _CH)], sbuf)

        def gbody(r, carry):
            sbuf[r, :] = 1.0 / jnp.maximum(sbuf[r, :], 1.0)
            return carry
        lax.fori_loop(0, _CH, gbody, 0)
        pltpu.sync_copy(sbuf, invc.at[c].at[pl.ds(ebase + k * _CH, _CH)])
    plsc.subcore_barrier()

    for bl in range(_NB // _NCORE):
        bg = c * (_NB // _NCORE) + bl
        zero_acc_e()
        pltpu.async_copy(sbuf, acc_v.at[pl.ds(s * 632, _CH)], sem_s)
        pltpu.async_copy(sbuf, acc_v.at[pl.ds(s * 632 + _CH, _CH)], sem_s)
        pltpu.async_copy(sbuf.at[pl.ds(0, 120)],
                         acc_v.at[pl.ds(s * 632 + 2 * _CH, 120)], sem_s)

        @pl.when(s < _NSUB - 1)
        def _():
            pltpu.async_copy(xh.at[bg].at[pl.ds(s * 632, 632)],
                             tbl.at[pl.ds(s * 632, 632)], sem_s)
            pltpu.make_async_copy(xh.at[bg].at[pl.ds(s * 632, 632)],
                                  tbl.at[pl.ds(s * 632, 632)], sem_s).wait()

        @pl.when(s == _NSUB - 1)
        def _():
            pltpu.async_copy(xh.at[bg].at[pl.ds(15 * 632, _NP - 15 * 632)],
                             tbl.at[pl.ds(15 * 632, _NP - 15 * 632)], sem_s)
            pltpu.make_async_copy(
                xh.at[bg].at[pl.ds(15 * 632, _NP - 15 * 632)],
                tbl.at[pl.ds(15 * 632, _NP - 15 * 632)], sem_s).wait()

        pltpu.make_async_copy(sbuf, acc_v.at[pl.ds(s * 632, _CH)],
                              sem_s).wait()
        pltpu.make_async_copy(sbuf, acc_v.at[pl.ds(s * 632, _CH)],
                              sem_s).wait()
        pltpu.make_async_copy(sbuf.at[pl.ds(0, 120)],
                              acc_v.at[pl.ds(s * 632, 120)], sem_s).wait()
        drain_zero_acc_e()
        plsc.subcore_barrier()

        run_pass(tbl, v2d, e2d, ivb, ieb, acc_e)
        plsc.subcore_barrier()

        for k in range(_ECH):
            pltpu.async_copy(acc_e.at[pl.ds(ebase + k * _CH, _CH)], sbuf,
                             sem_g)
            pltpu.async_copy(invc.at[c].at[pl.ds(ebase + k * _CH, _CH)],
                             jbuf, sem_s)
            pltpu.make_async_copy(acc_e.at[pl.ds(ebase, _CH)], sbuf,
                                  sem_g).wait()
            pltpu.make_async_copy(invc.at[c].at[pl.ds(ebase, _CH)], jbuf,
                                  sem_s).wait()

            def scbody(r, carry):
                sbuf[r, :] = sbuf[r, :] * jbuf[r, :]
                return carry
            lax.fori_loop(0, _CH, scbody, 0)
            pltpu.sync_copy(sbuf, acc_e.at[pl.ds(ebase + k * _CH, _CH)])
        plsc.subcore_barrier()

        run_pass(acc_e, e2d, v2d, ieb, ivb, acc_v)
        plsc.subcore_barrier()

        @pl.when(s < _NSUB - 1)
        def _():
            pltpu.sync_copy(acc_v.at[pl.ds(s * 624, 624)],
                            out.at[bg].at[pl.ds(s * 624, 624)])

        @pl.when(s == _NSUB - 1)
        def _():
            pltpu.sync_copy(acc_v.at[pl.ds(15 * 624, 640)],
                            out.at[bg].at[pl.ds(15 * 624, 640)])
        plsc.subcore_barrier()


_sc_call = pl.kernel(
    _sc_body,
    out_type=(
        jax.ShapeDtypeStruct((_NB, _N, _F), jnp.float32),
        jax.ShapeDtypeStruct((_NCORE, _ME, _F), jnp.float32),
    ),
    mesh=plsc.VectorSubcoreMesh(core_axis_name="c", subcore_axis_name="s"),
    compiler_params=pltpu.CompilerParams(use_tc_tiling_on_sc=False),
    scratch_types=[
        pltpu.VMEM_SHARED((_ME, _F), jnp.float32),
        pltpu.VMEM_SHARED((_NV, _F), jnp.float32),
        pltpu.VMEM_SHARED((_NP, _F), jnp.float32),
        pltpu.VMEM((2, _BATCH), jnp.int32),
        pltpu.VMEM((2, _BATCH), jnp.int32),
        pltpu.VMEM((2, _BATCH, _F), jnp.float32),
        pltpu.VMEM((_CH, _F), jnp.float32),
        pltpu.VMEM((_CH, _F), jnp.float32),
        pltpu.SemaphoreType.DMA,
        pltpu.SemaphoreType.DMA,
    ],
)


def kernel(X, vertex, edges, W, eps):
    Xh = _matmul(X, W)

    xh_pad = jnp.concatenate(
        [Xh, jnp.zeros((_NP - _N, _HID), jnp.float32)], axis=0)
    xh_b = xh_pad.reshape(_NP, _NB, _F).transpose(1, 0, 2)

    pad = _EPAD - _E
    v2d = jnp.concatenate(
        [vertex.astype(jnp.int32), jnp.full((pad,), _N, jnp.int32)]
    ).reshape(_EPAD // _BATCH, _BATCH)
    e2d = jnp.concatenate(
        [edges.astype(jnp.int32), jnp.full((pad,), _M, jnp.int32)]
    ).reshape(_EPAD // _BATCH, _BATCH)

    zsrc = jnp.zeros((_CH, _F), jnp.float32)
    osrc = jnp.ones((_BATCH, _F), jnp.float32)

    Xv_b, _unused_invc = _sc_call(xh_b, v2d, e2d, zsrc, osrc)
    Xv = Xv_b.transpose(1, 0, 2).reshape(_N, _HID)
    return _epilogue(eps, Xh, Xv)


# speedup vs baseline: 2.0131x; 1.6267x over previous
import functools
import jax
import jax.numpy as jnp
from jax import lax
from jax.experimental import pallas as pl
from jax.experimental.pallas import tpu as pltpu
from jax.experimental.pallas import tpu_sc as plsc

_N = 10000
_E = 320000
_M = 80000
_IN = 128
_HID = 256

_F = 16
_NB = _HID // _F
_NCORE = 2
_NSUB = 16
_BATCH = 512
_JTILE = 40
_EPAD = _JTILE * _NSUB * _BATCH
_ME = 81920
_CH = 256
_ECH = _ME // _NSUB // _CH
_NV = 10112
_NP = 10016


def _mm_kernel(x_ref, w_ref, o_ref):
    o_ref[...] = jnp.dot(x_ref[...], w_ref[...],
                         preferred_element_type=jnp.float32)


def _matmul(X, W):
    BM = 1000
    return pl.pallas_call(
        _mm_kernel,
        grid=(_N // BM,),
        in_specs=[
            pl.BlockSpec((BM, _IN), lambda i: (i, 0)),
            pl.BlockSpec((_IN, _HID), lambda i: (0, 0)),
        ],
        out_specs=pl.BlockSpec((BM, _HID), lambda i: (i, 0)),
        out_shape=jax.ShapeDtypeStruct((_N, _HID), jnp.float32),
    )(X, W)


def _ep_kernel(eps_ref, xh_ref, xv_ref, o_ref):
    o = (1.0 + eps_ref[0]) * xh_ref[...] + xv_ref[...]
    ss = jnp.sum(o * o, axis=1, keepdims=True)
    rn = jnp.sqrt(ss)
    scale = jnp.where(rn > 0, 1.0 / rn, 0.0)
    o_ref[...] = o * scale


def _epilogue(eps, Xh, Xv):
    BM = 1000
    return pl.pallas_call(
        _ep_kernel,
        grid=(_N // BM,),
        in_specs=[
            pl.BlockSpec(memory_space=pltpu.SMEM),
            pl.BlockSpec((BM, _HID), lambda i: (i, 0)),
            pl.BlockSpec((BM, _HID), lambda i: (i, 0)),
        ],
        out_specs=pl.BlockSpec((BM, _HID), lambda i: (i, 0)),
        out_shape=jax.ShapeDtypeStruct((_N, _HID), jnp.float32),
    )(eps, Xh, Xv)


def _sc_body(xh, v2d, e2d, zsrc, osrc, out, invc,
             acc_e, acc_v, tbl, ivb, ieb, rows, sbuf, jbuf,
             sem_g, sem_s, sem_i):
    c = lax.axis_index("c")
    s = lax.axis_index("s")

    ebase = s * (_ME // _NSUB)
    jbase = s * _JTILE

    def zero_acc_e():
        pltpu.sync_copy(zsrc, sbuf)
        for k in range(_ECH):
            pltpu.async_copy(sbuf, acc_e.at[pl.ds(ebase + k * _CH, _CH)],
                             sem_g)

    def drain_zero_acc_e():
        for k in range(_ECH):
            pltpu.make_async_copy(sbuf, acc_e.at[pl.ds(ebase, _CH)],
                                  sem_g).wait()

    zero_acc_e()
    drain_zero_acc_e()
    plsc.subcore_barrier()

    pltpu.sync_copy(osrc, rows.at[0])
    pltpu.sync_copy(e2d.at[jbase], ieb.at[0])

    def cbody(jj, carry):
        for b in (0, 1):
            j = 2 * jj + b

            @pl.when(jj >= 1)
            def _():
                pltpu.make_async_copy(rows.at[0], acc_e.at[ieb.at[b]],
                                      sem_s).wait()
            pltpu.async_copy(rows.at[0], acc_e.at[ieb.at[b]], sem_s, add=True)

            @pl.when(j < _JTILE - 1)
            def _():
                pltpu.sync_copy(e2d.at[jbase + j + 1], ieb.at[1 - b])
        return carry
    lax.fori_loop(0, _JTILE // 2, cbody, 0)
    pltpu.make_async_copy(rows.at[0], acc_e.at[ieb.at[0]], sem_s).wait()
    pltpu.make_async_copy(rows.at[0], acc_e.at[ieb.at[1]], sem_s).wait()
    plsc.subcore_barrier()

    def run_pass(gtable, gidx_hbm, sidx_hbm, gidx, sidx, dacc):
        def load_idx(j, sl):
            pltpu.async_copy(gidx_hbm.at[jbase + j], gidx.at[sl], sem_i)
            pltpu.async_copy(sidx_hbm.at[jbase + j], sidx.at[sl], sem_i)

        def wait_idx(j, sl):
            pltpu.make_async_copy(gidx_hbm.at[jbase + j], gidx.at[sl],
                                  sem_i).wait()
            pltpu.make_async_copy(sidx_hbm.at[jbase + j], sidx.at[sl],
                                  sem_i).wait()

        def issue_g(b, sl):
            pltpu.async_copy(gtable.at[gidx.at[sl]], rows.at[b], sem_g)

        def wait_g(b, sl):
            pltpu.make_async_copy(gtable.at[gidx.at[sl]], rows.at[b],
                                  sem_g).wait()

        def issue_s(b, sl):
            pltpu.async_copy(rows.at[b], dacc.at[sidx.at[sl]], sem_s,
                             add=True)

        def wait_s(b, sl):
            pltpu.make_async_copy(rows.at[b], dacc.at[sidx.at[sl]],
                                  sem_s).wait()

        def step(j, b, sl, first=False, prefetch=True):
            if not first:
                wait_s(b, sl)
            wait_idx(j, sl)
            issue_g(b, sl)
            if prefetch:
                load_idx(j + 2, (sl + 2) % 4)
            if not first:
                wait_g(1 - b, (sl + 3) % 4)
                issue_s(1 - b, (sl + 3) % 4)

        load_idx(0, 0)
        load_idx(1, 1)
        step(0, 0, 0, first=True)
        step(1, 1, 1, first=True)
        wait_g(0, 0)
        issue_s(0, 0)

        def pbody(jj, carry):
            for b2 in (0, 1, 2, 3):
                j = 4 * jj + 2 + b2
                step(j, (2 + b2) % 2, (2 + b2) % 4)
            return carry
        lax.fori_loop(0, (_JTILE - 4) // 4, pbody, 0)

        step(_JTILE - 2, 0, 2, prefetch=False)
        step(_JTILE - 1, 1, 3, prefetch=False)
        wait_g(1, 3)
        issue_s(1, 3)
        wait_s(0, 2)
        wait_s(1, 3)

    for k in range(_ECH):
        pltpu.sync_copy(acc_e.at[pl.ds(ebase + k * _CH, _CH)], sbuf)

        def gbody(r4, carry):
            for d in range(4):
                r = 4 * r4 + d
                sbuf[r, :] = 1.0 / jnp.maximum(sbuf[r, :], 1.0)
            return carry
        lax.fori_loop(0, _CH // 4, gbody, 0)
        pltpu.sync_copy(sbuf, invc.at[c].at[pl.ds(ebase + k * _CH, _CH)])
    plsc.subcore_barrier()

    def block_body(bl, bcarry):
        bg = c * (_NB // _NCORE) + bl
        zero_acc_e()
        pltpu.async_copy(sbuf, acc_v.at[pl.ds(s * 632, _CH)], sem_s)
        pltpu.async_copy(sbuf, acc_v.at[pl.ds(s * 632 + _CH, _CH)], sem_s)
        pltpu.async_copy(sbuf.at[pl.ds(0, 120)],
                         acc_v.at[pl.ds(s * 632 + 2 * _CH, 120)], sem_s)

        @pl.when(s < _NSUB - 1)
        def _():
            pltpu.async_copy(xh.at[bg].at[pl.ds(s * 632, 632)],
                             tbl.at[pl.ds(s * 632, 632)], sem_s)
            pltpu.make_async_copy(xh.at[bg].at[pl.ds(s * 632, 632)],
                                  tbl.at[pl.ds(s * 632, 632)], sem_s).wait()

        @pl.when(s == _NSUB - 1)
        def _():
            pltpu.async_copy(xh.at[bg].at[pl.ds(15 * 632, _NP - 15 * 632)],
                             tbl.at[pl.ds(15 * 632, _NP - 15 * 632)], sem_s)
            pltpu.make_async_copy(
                xh.at[bg].at[pl.ds(15 * 632, _NP - 15 * 632)],
                tbl.at[pl.ds(15 * 632, _NP - 15 * 632)], sem_s).wait()

        pltpu.make_async_copy(sbuf, acc_v.at[pl.ds(s * 632, _CH)],
                              sem_s).wait()
        pltpu.make_async_copy(sbuf, acc_v.at[pl.ds(s * 632, _CH)],
                              sem_s).wait()
        pltpu.make_async_copy(sbuf.at[pl.ds(0, 120)],
                              acc_v.at[pl.ds(s * 632, 120)], sem_s).wait()
        drain_zero_acc_e()
        plsc.subcore_barrier()

        run_pass(tbl, v2d, e2d, ivb, ieb, acc_e)
        plsc.subcore_barrier()

        for k in range(_ECH):
            pltpu.async_copy(acc_e.at[pl.ds(ebase + k * _CH, _CH)], sbuf,
                             sem_g)
            pltpu.async_copy(invc.at[c].at[pl.ds(ebase + k * _CH, _CH)],
                             jbuf, sem_s)
            pltpu.make_async_copy(acc_e.at[pl.ds(ebase, _CH)], sbuf,
                                  sem_g).wait()
            pltpu.make_async_copy(invc.at[c].at[pl.ds(ebase, _CH)], jbuf,
                                  sem_s).wait()

            def scbody(r4, carry):
                for d in range(4):
                    r = 4 * r4 + d
                    sbuf[r, :] = sbuf[r, :] * jbuf[r, :]
                return carry
            lax.fori_loop(0, _CH // 4, scbody, 0)
            pltpu.sync_copy(sbuf, acc_e.at[pl.ds(ebase + k * _CH, _CH)])
        plsc.subcore_barrier()

        run_pass(acc_e, e2d, v2d, ieb, ivb, acc_v)
        plsc.subcore_barrier()

        @pl.when(s < _NSUB - 1)
        def _():
            pltpu.sync_copy(acc_v.at[pl.ds(s * 624, 624)],
                            out.at[bg].at[pl.ds(s * 624, 624)])

        @pl.when(s == _NSUB - 1)
        def _():
            pltpu.sync_copy(acc_v.at[pl.ds(15 * 624, 640)],
                            out.at[bg].at[pl.ds(15 * 624, 640)])
        plsc.subcore_barrier()
        return bcarry

    lax.fori_loop(0, _NB // _NCORE, block_body, 0)


_sc_call = pl.kernel(
    _sc_body,
    out_type=(
        jax.ShapeDtypeStruct((_NB, _N, _F), jnp.float32),
        jax.ShapeDtypeStruct((_NCORE, _ME, _F), jnp.float32),
    ),
    mesh=plsc.VectorSubcoreMesh(core_axis_name="c", subcore_axis_name="s"),
    compiler_params=pltpu.CompilerParams(use_tc_tiling_on_sc=False),
    scratch_types=[
        pltpu.VMEM_SHARED((_ME, _F), jnp.float32),
        pltpu.VMEM_SHARED((_NV, _F), jnp.float32),
        pltpu.VMEM_SHARED((_NP, _F), jnp.float32),
        pltpu.VMEM((4, _BATCH), jnp.int32),
        pltpu.VMEM((4, _BATCH), jnp.int32),
        pltpu.VMEM((2, _BATCH, _F), jnp.float32),
        pltpu.VMEM((_CH, _F), jnp.float32),
        pltpu.VMEM((_CH, _F), jnp.float32),
        pltpu.SemaphoreType.DMA,
        pltpu.SemaphoreType.DMA,
        pltpu.SemaphoreType.DMA,
    ],
)


def kernel(X, vertex, edges, W, eps):
    Xh = _matmul(X, W)

    xh_pad = jnp.concatenate(
        [Xh, jnp.zeros((_NP - _N, _HID), jnp.float32)], axis=0)
    xh_b = xh_pad.reshape(_NP, _NB, _F).transpose(1, 0, 2)

    pad = _EPAD - _E
    v2d = jnp.concatenate(
        [vertex.astype(jnp.int32), jnp.full((pad,), _N, jnp.int32)]
    ).reshape(_EPAD // _BATCH, _BATCH)
    e2d = jnp.concatenate(
        [edges.astype(jnp.int32), jnp.full((pad,), _M, jnp.int32)]
    ).reshape(_EPAD // _BATCH, _BATCH)

    zsrc = jnp.zeros((_CH, _F), jnp.float32)
    osrc = jnp.ones((_BATCH, _F), jnp.float32)

    Xv_b, _unused_invc = _sc_call(xh_b, v2d, e2d, zsrc, osrc)
    Xv = Xv_b.transpose(1, 0, 2).reshape(_N, _HID)
    return _epilogue(eps, Xh, Xv)


# speedup vs baseline: 2.0145x; 1.0007x over previous
import functools
import jax
import jax.numpy as jnp
from jax import lax
from jax.experimental import pallas as pl
from jax.experimental.pallas import tpu as pltpu
from jax.experimental.pallas import tpu_sc as plsc

_N = 10000
_E = 320000
_M = 80000
_IN = 128
_HID = 256

_F = 16
_NB = _HID // _F
_NCORE = 2
_NSUB = 16
_BATCH = 512
_JTILE = 40
_EPAD = _JTILE * _NSUB * _BATCH
_ME = 81920
_CH = 256
_ECH = _ME // _NSUB // _CH
_NV = 10112
_NP = 10016


def _mm_kernel(x_ref, w_ref, o_ref):
    o_ref[...] = jnp.dot(x_ref[...], w_ref[...],
                         preferred_element_type=jnp.float32)


def _matmul(X, W):
    BM = 1000
    return pl.pallas_call(
        _mm_kernel,
        grid=(_N // BM,),
        in_specs=[
            pl.BlockSpec((BM, _IN), lambda i: (i, 0)),
            pl.BlockSpec((_IN, _HID), lambda i: (0, 0)),
        ],
        out_specs=pl.BlockSpec((BM, _HID), lambda i: (i, 0)),
        out_shape=jax.ShapeDtypeStruct((_N, _HID), jnp.float32),
    )(X, W)


def _ep_kernel(eps_ref, xh_ref, xv_ref, o_ref):
    o = (1.0 + eps_ref[0]) * xh_ref[...] + xv_ref[...]
    ss = jnp.sum(o * o, axis=1, keepdims=True)
    rn = jnp.sqrt(ss)
    scale = jnp.where(rn > 0, 1.0 / rn, 0.0)
    o_ref[...] = o * scale


def _epilogue(eps, Xh, Xv):
    BM = 1000
    return pl.pallas_call(
        _ep_kernel,
        grid=(_N // BM,),
        in_specs=[
            pl.BlockSpec(memory_space=pltpu.SMEM),
            pl.BlockSpec((BM, _HID), lambda i: (i, 0)),
            pl.BlockSpec((BM, _HID), lambda i: (i, 0)),
        ],
        out_specs=pl.BlockSpec((BM, _HID), lambda i: (i, 0)),
        out_shape=jax.ShapeDtypeStruct((_N, _HID), jnp.float32),
    )(eps, Xh, Xv)


def _sc_body(xh, v2d, e2d, zsrc, osrc, out, invc,
             acc_e, acc_v, tbl, ivb, ieb, rows, sbuf, jbuf,
             sem_g, sem_s, sem_i, sem_t):
    c = lax.axis_index("c")
    s = lax.axis_index("s")

    ebase = s * (_ME // _NSUB)
    jbase = s * _JTILE

    def zero_acc_e():
        pltpu.sync_copy(zsrc, sbuf)
        for k in range(_ECH):
            pltpu.async_copy(sbuf, acc_e.at[pl.ds(ebase + k * _CH, _CH)],
                             sem_g)

    def drain_zero_acc_e():
        for k in range(_ECH):
            pltpu.make_async_copy(sbuf, acc_e.at[pl.ds(ebase, _CH)],
                                  sem_g).wait()

    zero_acc_e()
    drain_zero_acc_e()
    plsc.subcore_barrier()

    pltpu.sync_copy(osrc, rows.at[0])

    def load_e(j, sl):
        pltpu.async_copy(e2d.at[jbase + j], ieb.at[sl], sem_i)

    def wait_e(j, sl):
        pltpu.make_async_copy(e2d.at[jbase + j], ieb.at[sl], sem_i).wait()

    def issue_cs(sl):
        pltpu.async_copy(rows.at[0], acc_e.at[ieb.at[sl]], sem_s, add=True)

    def wait_cs(sl):
        pltpu.make_async_copy(rows.at[0], acc_e.at[ieb.at[sl]], sem_s).wait()

    load_e(0, 0)
    load_e(1, 1)
    wait_e(0, 0)
    issue_cs(0)
    load_e(2, 2)
    wait_e(1, 1)
    issue_cs(1)
    load_e(3, 3)

    def cbody(jj, carry):
        for b2 in (0, 1, 2, 3):
            j = 4 * jj + 2 + b2
            sl = (2 + b2) % 4
            wait_cs((sl + 2) % 4)
            wait_e(j, sl)
            issue_cs(sl)
            load_e(j + 2, (sl + 2) % 4)
        return carry
    lax.fori_loop(0, (_JTILE - 4) // 4, cbody, 0)
    wait_cs(0)
    wait_e(_JTILE - 2, 2)
    issue_cs(2)
    wait_cs(1)
    wait_e(_JTILE - 1, 3)
    issue_cs(3)
    wait_cs(2)
    wait_cs(3)
    plsc.subcore_barrier()

    def run_pass(gtable, gidx_hbm, sidx_hbm, gidx, sidx, dacc):
        def load_idx(j, sl):
            pltpu.async_copy(gidx_hbm.at[jbase + j], gidx.at[sl], sem_i)
            pltpu.async_copy(sidx_hbm.at[jbase + j], sidx.at[sl], sem_i)

        def wait_idx(j, sl):
            pltpu.make_async_copy(gidx_hbm.at[jbase + j], gidx.at[sl],
                                  sem_i).wait()
            pltpu.make_async_copy(sidx_hbm.at[jbase + j], sidx.at[sl],
                                  sem_i).wait()

        def issue_g(b, sl):
            pltpu.async_copy(gtable.at[gidx.at[sl]], rows.at[b], sem_g)

        def wait_g(b, sl):
            pltpu.make_async_copy(gtable.at[gidx.at[sl]], rows.at[b],
                                  sem_g).wait()

        def issue_s(b, sl):
            pltpu.async_copy(rows.at[b], dacc.at[sidx.at[sl]], sem_s,
                             add=True)

        def wait_s(b, sl):
            pltpu.make_async_copy(rows.at[b], dacc.at[sidx.at[sl]],
                                  sem_s).wait()

        def step(j, b, sl, first=False, prefetch=True):
            if not first:
                wait_s(b, sl)
            wait_idx(j, sl)
            issue_g(b, sl)
            if prefetch:
                load_idx(j + 2, (sl + 2) % 4)
            if not first:
                wait_g(1 - b, (sl + 3) % 4)
                issue_s(1 - b, (sl + 3) % 4)

        load_idx(0, 0)
        load_idx(1, 1)
        step(0, 0, 0, first=True)
        step(1, 1, 1, first=True)
        wait_g(0, 0)
        issue_s(0, 0)

        def pbody(jj, carry):
            for b2 in (0, 1, 2, 3):
                j = 4 * jj + 2 + b2
                step(j, (2 + b2) % 2, (2 + b2) % 4)
            return carry
        lax.fori_loop(0, (_JTILE - 4) // 4, pbody, 0)

        step(_JTILE - 2, 0, 2, prefetch=False)
        step(_JTILE - 1, 1, 3, prefetch=False)
        wait_g(1, 3)
        issue_s(1, 3)
        wait_s(0, 2)
        wait_s(1, 3)

    for k in range(_ECH):
        pltpu.sync_copy(acc_e.at[pl.ds(ebase + k * _CH, _CH)], sbuf)

        def gbody(r4, carry):
            for d in range(4):
                r = 4 * r4 + d
                sbuf[r, :] = 1.0 / jnp.maximum(sbuf[r, :], 1.0)
            return carry
        lax.fori_loop(0, _CH // 4, gbody, 0)
        pltpu.sync_copy(sbuf, invc.at[c].at[pl.ds(ebase + k * _CH, _CH)])
    plsc.subcore_barrier()

    def stage_tbl(bgx):
        @pl.when(s < _NSUB - 1)
        def _():
            pltpu.async_copy(xh.at[bgx].at[pl.ds(s * 632, 632)],
                             tbl.at[pl.ds(s * 632, 632)], sem_t)

        @pl.when(s == _NSUB - 1)
        def _():
            pltpu.async_copy(xh.at[bgx].at[pl.ds(15 * 632, _NP - 15 * 632)],
                             tbl.at[pl.ds(15 * 632, _NP - 15 * 632)], sem_t)

    def wait_tbl(bgx):
        @pl.when(s < _NSUB - 1)
        def _():
            pltpu.make_async_copy(xh.at[bgx].at[pl.ds(s * 632, 632)],
                                  tbl.at[pl.ds(s * 632, 632)], sem_t).wait()

        @pl.when(s == _NSUB - 1)
        def _():
            pltpu.make_async_copy(
                xh.at[bgx].at[pl.ds(15 * 632, _NP - 15 * 632)],
                tbl.at[pl.ds(15 * 632, _NP - 15 * 632)], sem_t).wait()

    stage_tbl(c * (_NB // _NCORE))

    def block_body(bl, bcarry):
        bg = c * (_NB // _NCORE) + bl
        zero_acc_e()
        pltpu.async_copy(sbuf, acc_v.at[pl.ds(s * 632, _CH)], sem_s)
        pltpu.async_copy(sbuf, acc_v.at[pl.ds(s * 632 + _CH, _CH)], sem_s)
        pltpu.async_copy(sbuf.at[pl.ds(0, 120)],
                         acc_v.at[pl.ds(s * 632 + 2 * _CH, 120)], sem_s)

        wait_tbl(bg)
        pltpu.make_async_copy(sbuf, acc_v.at[pl.ds(s * 632, _CH)],
                              sem_s).wait()
        pltpu.make_async_copy(sbuf, acc_v.at[pl.ds(s * 632, _CH)],
                              sem_s).wait()
        pltpu.make_async_copy(sbuf.at[pl.ds(0, 120)],
                              acc_v.at[pl.ds(s * 632, 120)], sem_s).wait()
        drain_zero_acc_e()
        plsc.subcore_barrier()

        run_pass(tbl, v2d, e2d, ivb, ieb, acc_e)
        plsc.subcore_barrier()

        @pl.when(bl < _NB // _NCORE - 1)
        def _():
            stage_tbl(bg + 1)

        for k in range(_ECH):
            pltpu.async_copy(acc_e.at[pl.ds(ebase + k * _CH, _CH)], sbuf,
                             sem_g)
            pltpu.async_copy(invc.at[c].at[pl.ds(ebase + k * _CH, _CH)],
                             jbuf, sem_s)
            pltpu.make_async_copy(acc_e.at[pl.ds(ebase, _CH)], sbuf,
                                  sem_g).wait()
            pltpu.make_async_copy(invc.at[c].at[pl.ds(ebase, _CH)], jbuf,
                                  sem_s).wait()

            def scbody(r4, carry):
                for d in range(4):
                    r = 4 * r4 + d
                    sbuf[r, :] = sbuf[r, :] * jbuf[r, :]
                return carry
            lax.fori_loop(0, _CH // 4, scbody, 0)
            pltpu.sync_copy(sbuf, acc_e.at[pl.ds(ebase + k * _CH, _CH)])
        plsc.subcore_barrier()

        run_pass(acc_e, e2d, v2d, ieb, ivb, acc_v)
        plsc.subcore_barrier()

        @pl.when(s < _NSUB - 1)
        def _():
            pltpu.sync_copy(acc_v.at[pl.ds(s * 624, 624)],
                            out.at[bg].at[pl.ds(s * 624, 624)])

        @pl.when(s == _NSUB - 1)
        def _():
            pltpu.sync_copy(acc_v.at[pl.ds(15 * 624, 640)],
                            out.at[bg].at[pl.ds(15 * 624, 640)])
        plsc.subcore_barrier()
        return bcarry

    lax.fori_loop(0, _NB // _NCORE, block_body, 0)


_sc_call = pl.kernel(
    _sc_body,
    out_type=(
        jax.ShapeDtypeStruct((_NB, _N, _F), jnp.float32),
        jax.ShapeDtypeStruct((_NCORE, _ME, _F), jnp.float32),
    ),
    mesh=plsc.VectorSubcoreMesh(core_axis_name="c", subcore_axis_name="s"),
    compiler_params=pltpu.CompilerParams(use_tc_tiling_on_sc=False),
    scratch_types=[
        pltpu.VMEM_SHARED((_ME, _F), jnp.float32),
        pltpu.VMEM_SHARED((_NV, _F), jnp.float32),
        pltpu.VMEM_SHARED((_NP, _F), jnp.float32),
        pltpu.VMEM((4, _BATCH), jnp.int32),
        pltpu.VMEM((4, _BATCH), jnp.int32),
        pltpu.VMEM((2, _BATCH, _F), jnp.float32),
        pltpu.VMEM((_CH, _F), jnp.float32),
        pltpu.VMEM((_CH, _F), jnp.float32),
        pltpu.SemaphoreType.DMA,
        pltpu.SemaphoreType.DMA,
        pltpu.SemaphoreType.DMA,
        pltpu.SemaphoreType.DMA,
    ],
)


def kernel(X, vertex, edges, W, eps):
    Xh = _matmul(X, W)

    xh_pad = jnp.concatenate(
        [Xh, jnp.zeros((_NP - _N, _HID), jnp.float32)], axis=0)
    xh_b = xh_pad.reshape(_NP, _NB, _F).transpose(1, 0, 2)

    pad = _EPAD - _E
    v2d = jnp.concatenate(
        [vertex.astype(jnp.int32), jnp.full((pad,), _N, jnp.int32)]
    ).reshape(_EPAD // _BATCH, _BATCH)
    e2d = jnp.concatenate(
        [edges.astype(jnp.int32), jnp.full((pad,), _M, jnp.int32)]
    ).reshape(_EPAD // _BATCH, _BATCH)

    zsrc = jnp.zeros((_CH, _F), jnp.float32)
    osrc = jnp.ones((_BATCH, _F), jnp.float32)

    Xv_b, _unused_invc = _sc_call(xh_b, v2d, e2d, zsrc, osrc)
    Xv = Xv_b.transpose(1, 0, 2).reshape(_N, _HID)
    return _epilogue(eps, Xh, Xv)
